# baseline (speedup 1.0000x reference)
import functools

import jax
import jax.numpy as jnp
from jax import lax
from jax.experimental import pallas as pl
from jax.experimental.pallas import tpu as pltpu
from jax.experimental.pallas import tpu_sc as plsc

EMBED_DIM = 32


@functools.partial(jax.jit, static_argnums=(2, 3, 4))
def _gather_rows_t(idx, table, BATCH, HIST, NC):
    mesh = plsc.VectorSubcoreMesh(core_axis_name="c", subcore_axis_name="s")
    NW = NC * 16
    W = BATCH // NW

    @functools.partial(
        pl.kernel,
        mesh=mesh,
        out_type=jax.ShapeDtypeStruct((HIST, EMBED_DIM, BATCH), jnp.float32),
        scratch_types=[
            pltpu.VMEM((W,), jnp.int32),
            pltpu.VMEM((W,), jnp.int32),
            pltpu.VMEM((W, EMBED_DIM), jnp.float32),
            pltpu.VMEM((W, EMBED_DIM), jnp.float32),
            pltpu.VMEM((EMBED_DIM, W), jnp.float32),
            pltpu.SemaphoreType.DMA,
            pltpu.SemaphoreType.DMA,
        ],
        compiler_params=pltpu.CompilerParams(
            use_tc_tiling_on_sc=False, needs_layout_passes=False
        ),
    )
    def k(idx_hbm, table_hbm, out_hbm, idx0, idx1, rows0, rows1, cols, g0, g1):
        wid = lax.axis_index("s") * NC + lax.axis_index("c")
        b0 = wid * W
        i16 = lax.iota(jnp.int32, 16)

        def load_and_gather(h, idx_v, rows_v, g):
            base = pl.multiple_of(h * BATCH + b0, 8)
            pltpu.sync_copy(idx_hbm.at[pl.ds(base, W)], idx_v)
            pltpu.async_copy(table_hbm.at[idx_v], rows_v, g)

        def wait_gather(idx_v, rows_v, g):
            pltpu.make_async_copy(table_hbm.at[idx_v], rows_v, g).wait()

        def transpose_store(h, rows_v):
            def dbody(d, carry):
                d16 = jnp.full((16,), 0, jnp.int32) + d
                for g in range(W // 16):
                    v = plsc.load_gather(rows_v, [g * 16 + i16, d16])
                    cols[d, pl.ds(g * 16, 16)] = v
                return carry

            lax.fori_loop(0, EMBED_DIM, dbody, 0)
            pltpu.sync_copy(cols, out_hbm.at[h, :, pl.ds(b0, W)])

        load_and_gather(0, idx0, rows0, g0)
        load_and_gather(1, idx1, rows1, g1)

        def body(j, carry):
            a = 2 * j
            wait_gather(idx0, rows0, g0)
            transpose_store(a, rows0)

            @pl.when(j < HIST // 2 - 1)
            def _():
                load_and_gather(a + 2, idx0, rows0, g0)

            wait_gather(idx1, rows1, g1)
            transpose_store(a + 1, rows1)

            @pl.when(j < HIST // 2 - 1)
            def _():
                load_and_gather(a + 3, idx1, rows1, g1)

            return carry

        lax.fori_loop(0, HIST // 2, body, 0)

    return k(idx, table)


def kernel(x, w):
    BATCH, HIST = x.shape
    B = BATCH * HIST
    idx = jnp.swapaxes(x, 0, 1).reshape(B).astype(jnp.int32)
    info = plsc.get_sparse_core_info()
    NC = info.num_cores
    out = _gather_rows_t(idx, w, BATCH, HIST, NC)
    return jnp.transpose(out, (2, 0, 1))


# speedup vs baseline: 1.4777x; 1.4777x over previous
import functools

import jax
import jax.numpy as jnp
from jax import lax
from jax.experimental import pallas as pl
from jax.experimental.pallas import tpu as pltpu
from jax.experimental.pallas import tpu_sc as plsc

EMBED_DIM = 32


@functools.partial(jax.jit, static_argnums=(2, 3, 4))
def _gather_rows_t(idx, table, BATCH, HIST, NC):
    mesh = plsc.VectorSubcoreMesh(core_axis_name="c", subcore_axis_name="s")
    NW = NC * 16
    W = BATCH // NW

    @functools.partial(
        pl.kernel,
        mesh=mesh,
        out_type=jax.ShapeDtypeStruct((HIST, EMBED_DIM, BATCH), jnp.float32),
        scratch_types=[
            pltpu.VMEM((W,), jnp.int32),
            pltpu.VMEM((W,), jnp.int32),
            pltpu.VMEM((W, EMBED_DIM), jnp.float32),
            pltpu.VMEM((W, EMBED_DIM), jnp.float32),
            pltpu.VMEM((EMBED_DIM, W), jnp.float32),
            pltpu.SemaphoreType.DMA,
            pltpu.SemaphoreType.DMA,
        ],
        compiler_params=pltpu.CompilerParams(
            use_tc_tiling_on_sc=False, needs_layout_passes=False
        ),
    )
    def k(idx_hbm, table_hbm, out_hbm, idx0, idx1, rows0, rows1, cols, g0, g1):
        wid = lax.axis_index("s") * NC + lax.axis_index("c")
        b0 = wid * W
        i16 = lax.iota(jnp.int32, 16)

        def load_and_gather(h, idx_v, rows_v, g):
            base = pl.multiple_of(h * BATCH + b0, 8)
            pltpu.sync_copy(idx_hbm.at[pl.ds(base, W)], idx_v)
            pltpu.async_copy(table_hbm.at[idx_v], rows_v, g)

        def wait_gather(idx_v, rows_v, g):
            pltpu.make_async_copy(table_hbm.at[idx_v], rows_v, g).wait()

        def transpose_store(h, rows_v):
            def dbody(d, carry):
                dcol = (d + i16) & (EMBED_DIM - 1)
                for g in range(W // 16):
                    rowv = g * 16 + i16
                    v = plsc.load_gather(rows_v, [rowv, dcol])
                    plsc.store_scatter(cols, [dcol, rowv], v)
                return carry

            lax.fori_loop(0, EMBED_DIM, dbody, 0)
            pltpu.sync_copy(cols, out_hbm.at[h, :, pl.ds(b0, W)])

        load_and_gather(0, idx0, rows0, g0)
        load_and_gather(1, idx1, rows1, g1)

        def body(j, carry):
            a = 2 * j
            wait_gather(idx0, rows0, g0)
            transpose_store(a, rows0)

            @pl.when(j < HIST // 2 - 1)
            def _():
                load_and_gather(a + 2, idx0, rows0, g0)

            wait_gather(idx1, rows1, g1)
            transpose_store(a + 1, rows1)

            @pl.when(j < HIST // 2 - 1)
            def _():
                load_and_gather(a + 3, idx1, rows1, g1)

            return carry

        lax.fori_loop(0, HIST // 2, body, 0)

    return k(idx, table)


def kernel(x, w):
    BATCH, HIST = x.shape
    B = BATCH * HIST
    idx = jnp.swapaxes(x, 0, 1).reshape(B).astype(jnp.int32)
    info = plsc.get_sparse_core_info()
    NC = info.num_cores
    out = _gather_rows_t(idx, w, BATCH, HIST, NC)
    return jnp.transpose(out, (2, 0, 1))


# speedup vs baseline: 1.5264x; 1.0329x over previous
import functools

import jax
import jax.numpy as jnp
from jax import lax
from jax.experimental import pallas as pl
from jax.experimental.pallas import tpu as pltpu
from jax.experimental.pallas import tpu_sc as plsc

EMBED_DIM = 32


@functools.partial(jax.jit, static_argnums=(2, 3, 4))
def _gather_rows_t(idx, table, BATCH, HIST, NC):
    mesh = plsc.VectorSubcoreMesh(core_axis_name="c", subcore_axis_name="s")
    NW = NC * 16
    W = BATCH // NW

    @functools.partial(
        pl.kernel,
        mesh=mesh,
        out_type=jax.ShapeDtypeStruct((HIST, EMBED_DIM, BATCH), jnp.float32),
        scratch_types=[
            pltpu.VMEM((W,), jnp.int32),
            pltpu.VMEM((W,), jnp.int32),
            pltpu.VMEM((W, EMBED_DIM), jnp.float32),
            pltpu.VMEM((W, EMBED_DIM), jnp.float32),
            pltpu.VMEM((EMBED_DIM, W), jnp.float32),
            pltpu.VMEM((EMBED_DIM, W), jnp.float32),
            pltpu.SemaphoreType.DMA,
            pltpu.SemaphoreType.DMA,
            pltpu.SemaphoreType.DMA,
            pltpu.SemaphoreType.DMA,
        ],
        compiler_params=pltpu.CompilerParams(
            use_tc_tiling_on_sc=False, needs_layout_passes=False
        ),
    )
    def k(idx_hbm, table_hbm, out_hbm, idx0, idx1, rows0, rows1, cols0, cols1,
          g0, g1, s0, s1):
        wid = lax.axis_index("s") * NC + lax.axis_index("c")
        b0 = wid * W
        i16 = lax.iota(jnp.int32, 16)
        rowvs = [g * 16 + i16 for g in range(W // 16)]

        def load_and_gather(h, idx_v, rows_v, g):
            base = pl.multiple_of(h * BATCH + b0, 8)
            pltpu.sync_copy(idx_hbm.at[pl.ds(base, W)], idx_v)
            pltpu.async_copy(table_hbm.at[idx_v], rows_v, g)

        def wait_gather(idx_v, rows_v, g):
            pltpu.make_async_copy(table_hbm.at[idx_v], rows_v, g).wait()

        def transpose_store(h, rows_v, cols, s):
            def dbody(d, carry):
                dcol = (d + i16) & (EMBED_DIM - 1)
                for rowv in rowvs:
                    v = plsc.load_gather(rows_v, [rowv, dcol])
                    plsc.store_scatter(cols, [dcol, rowv], v)
                return carry

            lax.fori_loop(0, EMBED_DIM, dbody, 0)
            pltpu.async_copy(cols, out_hbm.at[h, :, pl.ds(b0, W)], s)

        def wait_store(h, cols, s):
            pltpu.make_async_copy(
                cols, out_hbm.at[h, :, pl.ds(b0, W)], s
            ).wait()

        load_and_gather(0, idx0, rows0, g0)
        load_and_gather(1, idx1, rows1, g1)

        def body(j, carry):
            a = 2 * j
            wait_gather(idx0, rows0, g0)

            @pl.when(j > 0)
            def _():
                wait_store(a - 2, cols0, s0)

            transpose_store(a, rows0, cols0, s0)

            @pl.when(j < HIST // 2 - 1)
            def _():
                load_and_gather(a + 2, idx0, rows0, g0)

            wait_gather(idx1, rows1, g1)

            @pl.when(j > 0)
            def _():
                wait_store(a - 1, cols1, s1)

            transpose_store(a + 1, rows1, cols1, s1)

            @pl.when(j < HIST // 2 - 1)
            def _():
                load_and_gather(a + 3, idx1, rows1, g1)

            return carry

        lax.fori_loop(0, HIST // 2, body, 0)
        wait_store(HIST - 2, cols0, s0)
        wait_store(HIST - 1, cols1, s1)

    return k(idx, table)


def kernel(x, w):
    BATCH, HIST = x.shape
    B = BATCH * HIST
    idx = jnp.swapaxes(x, 0, 1).reshape(B).astype(jnp.int32)
    info = plsc.get_sparse_core_info()
    NC = info.num_cores
    out = _gather_rows_t(idx, w, BATCH, HIST, NC)
    return jnp.transpose(out, (2, 0, 1))


# speedup vs baseline: 1.5303x; 1.0026x over previous
import functools

import jax
import jax.numpy as jnp
from jax import lax
from jax.experimental import pallas as pl
from jax.experimental.pallas import tpu as pltpu
from jax.experimental.pallas import tpu_sc as plsc

EMBED_DIM = 32


@functools.partial(jax.jit, static_argnums=(2, 3, 4))
def _gather_rows_t(idx, table, BATCH, HIST, NC):
    mesh = plsc.VectorSubcoreMesh(core_axis_name="c", subcore_axis_name="s")
    NW = NC * 16
    W = BATCH // NW

    @functools.partial(
        pl.kernel,
        mesh=mesh,
        out_type=jax.ShapeDtypeStruct((HIST, EMBED_DIM, BATCH), jnp.float32),
        scratch_types=[
            pltpu.VMEM((W,), jnp.int32),
            pltpu.VMEM((W,), jnp.int32),
            pltpu.VMEM((W, EMBED_DIM), jnp.float32),
            pltpu.VMEM((W, EMBED_DIM), jnp.float32),
            pltpu.VMEM((EMBED_DIM, W), jnp.float32),
            pltpu.VMEM((EMBED_DIM, W), jnp.float32),
            pltpu.SemaphoreType.DMA,
            pltpu.SemaphoreType.DMA,
            pltpu.SemaphoreType.DMA,
            pltpu.SemaphoreType.DMA,
        ],
        compiler_params=pltpu.CompilerParams(
            use_tc_tiling_on_sc=False, needs_layout_passes=False
        ),
    )
    def k(idx_hbm, table_hbm, out_hbm, idx0, idx1, rows0, rows1, cols0, cols1,
          g0, g1, s0, s1):
        wid = lax.axis_index("s") * NC + lax.axis_index("c")
        b0 = wid * W
        i16 = lax.iota(jnp.int32, 16)
        rowvs = [g * 16 + i16 for g in range(W // 16)]

        def load_and_gather(h, idx_v, rows_v, g):
            base = pl.multiple_of(h * BATCH + b0, 8)
            pltpu.sync_copy(idx_hbm.at[pl.ds(base, W)], idx_v)
            pltpu.async_copy(table_hbm.at[idx_v], rows_v, g)

        def wait_gather(idx_v, rows_v, g):
            pltpu.make_async_copy(table_hbm.at[idx_v], rows_v, g).wait()

        def transpose_store(h, rows_v, cols, s):
            def dbody(d4, carry):
                for u in range(4):
                    dcol = (d4 * 4 + u + i16) & (EMBED_DIM - 1)
                    for rowv in rowvs:
                        v = plsc.load_gather(rows_v, [rowv, dcol])
                        plsc.store_scatter(cols, [dcol, rowv], v)
                return carry

            lax.fori_loop(0, EMBED_DIM // 4, dbody, 0)
            pltpu.async_copy(cols, out_hbm.at[h, :, pl.ds(b0, W)], s)

        def wait_store(h, cols, s):
            pltpu.make_async_copy(
                cols, out_hbm.at[h, :, pl.ds(b0, W)], s
            ).wait()

        load_and_gather(0, idx0, rows0, g0)
        load_and_gather(1, idx1, rows1, g1)

        def body(j, carry):
            a = 2 * j
            wait_gather(idx0, rows0, g0)

            @pl.when(j > 0)
            def _():
                wait_store(a - 2, cols0, s0)

            transpose_store(a, rows0, cols0, s0)

            @pl.when(j < HIST // 2 - 1)
            def _():
                load_and_gather(a + 2, idx0, rows0, g0)

            wait_gather(idx1, rows1, g1)

            @pl.when(j > 0)
            def _():
                wait_store(a - 1, cols1, s1)

            transpose_store(a + 1, rows1, cols1, s1)

            @pl.when(j < HIST // 2 - 1)
            def _():
                load_and_gather(a + 3, idx1, rows1, g1)

            return carry

        lax.fori_loop(0, HIST // 2, body, 0)
        wait_store(HIST - 2, cols0, s0)
        wait_store(HIST - 1, cols1, s1)

    return k(idx, table)


def kernel(x, w):
    BATCH, HIST = x.shape
    B = BATCH * HIST
    idx = jnp.swapaxes(x, 0, 1).reshape(B).astype(jnp.int32)
    info = plsc.get_sparse_core_info()
    NC = info.num_cores
    out = _gather_rows_t(idx, w, BATCH, HIST, NC)
    return jnp.transpose(out, (2, 0, 1))
